# Initial kernel scaffold; baseline (speedup 1.0000x reference)
import jax
import jax.numpy as jnp
from jax.experimental import pallas as pl


def kernel(token_ids, parent_raw, emb, W_iou, U_iou, b_iou, W_f, U_f, b_f):
    raise NotImplementedError("write your pallas kernel here")


# speedup vs baseline: 1.2805x; 1.2805x over previous
import functools

import jax
import jax.numpy as jnp
from jax import lax
from jax.experimental import pallas as pl
from jax.experimental.pallas import tpu as pltpu
from jax.experimental.pallas import tpu_sc as plsc

H = 128
NC, NS = 2, 16
NW = NC * NS

LEVEL_SIZES = [64, 512, 4096, 45328]
STARTS = [0, 64, 576, 4672]

PAR_P = 4864
P3 = 46592
B_G = 53248

_mesh = plsc.VectorSubcoreMesh(
    core_axis_name="c", subcore_axis_name="s", num_cores=NC, num_subcores=NS)


def _make_gather(n_rows, n_chunks, ch, table_shape):
    assert n_rows == NW * n_chunks * ch

    @functools.partial(
        pl.kernel, mesh=_mesh,
        out_type=jax.ShapeDtypeStruct((n_rows, H), jnp.float32),
        scratch_types=[
            pltpu.VMEM((ch,), jnp.int32),
            pltpu.VMEM((ch, H), jnp.float32),
            pltpu.SemaphoreType.DMA,
        ],
    )
    def gather_k(idx_hbm, table_hbm, out_hbm, idx_v, rows_v, sem):
        wid = lax.axis_index("c") * NS + lax.axis_index("s")
        base = wid * (n_chunks * ch)
        for j in range(n_chunks):
            off = base + j * ch
            pltpu.sync_copy(idx_hbm.at[pl.ds(off, ch)], idx_v)
            pltpu.async_copy(table_hbm.at[idx_v], rows_v, sem).wait()
            pltpu.sync_copy(rows_v, out_hbm.at[pl.ds(off, ch)])

    return gather_k


def _make_scatter(n_rows, n_chunks, ch, n_seg, sp_rows):
    assert n_rows == NW * n_chunks * ch
    assert sp_rows % (8 * NS) == 0 and n_seg % 8 == 0
    zr = sp_rows // NS
    out_tiles = min(NS, n_seg // 8)
    orows = n_seg // out_tiles

    @functools.partial(
        pl.kernel, mesh=_mesh,
        out_type=(jax.ShapeDtypeStruct((NC, n_seg, H), jnp.float32),
                  jax.ShapeDtypeStruct((NC, n_seg, H), jnp.float32)),
        scratch_types=[
            pltpu.VMEM((ch,), jnp.int32),
            pltpu.VMEM((ch, H), jnp.float32),
            pltpu.VMEM_SHARED((sp_rows, H), jnp.float32),
            pltpu.VMEM_SHARED((sp_rows, H), jnp.float32),
        ],
    )
    def scatter_k(idx_hbm, h_hbm, fc_hbm, zeros_hbm, hs_out, fcs_out,
                  idx_v, row_v, hsum_sh, fcsum_sh):
        c = lax.axis_index("c")
        s = lax.axis_index("s")
        pltpu.sync_copy(zeros_hbm.at[pl.ds(s * zr, zr)], hsum_sh.at[pl.ds(s * zr, zr)])
        pltpu.sync_copy(zeros_hbm.at[pl.ds(s * zr, zr)], fcsum_sh.at[pl.ds(s * zr, zr)])
        plsc.subcore_barrier()
        base = (c * NS + s) * (n_chunks * ch)
        for j in range(n_chunks):
            off = base + j * ch
            pltpu.sync_copy(idx_hbm.at[pl.ds(off, ch)], idx_v)
            pltpu.sync_copy(h_hbm.at[pl.ds(off, ch)], row_v)
            pltpu.sync_copy(row_v, hsum_sh.at[idx_v], add=True)
            pltpu.sync_copy(fc_hbm.at[pl.ds(off, ch)], row_v)
            pltpu.sync_copy(row_v, fcsum_sh.at[idx_v], add=True)
        plsc.subcore_barrier()

        @pl.when(s < out_tiles)
        def _():
            pltpu.sync_copy(hsum_sh.at[pl.ds(s * orows, orows)],
                            hs_out.at[c, pl.ds(s * orows, orows)])
            pltpu.sync_copy(fcsum_sh.at[pl.ds(s * orows, orows)],
                            fcs_out.at[c, pl.ds(s * orows, orows)])

    return scatter_k


def _xw_body(x_ref, wiou_ref, biou_ref, wf_ref, bf_ref, xiou_ref, xwf_ref):
    x = x_ref[...]
    xiou_ref[...] = jnp.dot(x, wiou_ref[...], preferred_element_type=jnp.float32) + biou_ref[...]
    xwf_ref[...] = jnp.dot(x, wf_ref[...], preferred_element_type=jnp.float32) + bf_ref[...]


def _cell3_body(x_ref, wiou_ref, biou_ref, h_ref, c_ref):
    iou = jnp.dot(x_ref[...], wiou_ref[...], preferred_element_type=jnp.float32) + biou_ref[...]
    i = jax.nn.sigmoid(iou[:, :H])
    o = jax.nn.sigmoid(iou[:, H:2 * H])
    u = jnp.tanh(iou[:, 2 * H:])
    c = i * u
    c_ref[...] = c
    h_ref[...] = o * jnp.tanh(c)


def _f_body(xpf_ref, h_ref, c_ref, uf_ref, fc_ref):
    pre = xpf_ref[...] + jnp.dot(h_ref[...], uf_ref[...], preferred_element_type=jnp.float32)
    fc_ref[...] = jax.nn.sigmoid(pre) * c_ref[...]


def _cell_body(xiou_ref, hs_ref, fcs_ref, uiou_ref, h_ref, c_ref):
    h_sum = hs_ref[0] + hs_ref[1]
    iou = xiou_ref[...] + jnp.dot(h_sum, uiou_ref[...], preferred_element_type=jnp.float32)
    i = jax.nn.sigmoid(iou[:, :H])
    o = jax.nn.sigmoid(iou[:, H:2 * H])
    u = jnp.tanh(iou[:, 2 * H:])
    c = i * u + (fcs_ref[0] + fcs_ref[1])
    c_ref[...] = c
    h_ref[...] = o * jnp.tanh(c)


def _rep(shape):
    return pl.BlockSpec(shape, lambda i: tuple(0 for _ in shape))


def kernel(token_ids, parent_raw, emb, W_iou, U_iou, b_iou, W_f, U_f, b_f):
    f32 = jnp.float32
    token_ids = token_ids.astype(jnp.int32)
    parent_raw = parent_raw.astype(jnp.int32)
    b_iou2 = b_iou.reshape(1, 3 * H)
    b_f2 = b_f.reshape(1, H)

    tid = jnp.concatenate([
        token_ids[:4672], jnp.zeros((PAR_P - 4672,), jnp.int32),
        token_ids[4672:], jnp.zeros((B_G - PAR_P - 45328,), jnp.int32)])
    pad2 = P3 - 45328
    par2 = parent_raw[4672:50000] % 4096
    par2_g = jnp.concatenate([576 + par2, jnp.full((pad2,), 4672, jnp.int32)])
    par2_s = jnp.concatenate([par2, jnp.full((pad2,), 4096, jnp.int32)])
    par1 = parent_raw[576:4672] % 512
    par0 = parent_raw[64:576] % 64
    zeros_sp = jnp.zeros((4224, H), f32)

    x_buf = _make_gather(B_G, 13, 128, emb.shape)(tid, emb)

    xiou_par, xwf_par = pl.pallas_call(
        _xw_body,
        grid=(PAR_P // 128,),
        in_specs=[pl.BlockSpec((128, H), lambda i: (i, 0)),
                  _rep((H, 3 * H)), _rep((1, 3 * H)),
                  _rep((H, H)), _rep((1, H))],
        out_specs=[pl.BlockSpec((128, 3 * H), lambda i: (i, 0)),
                   pl.BlockSpec((128, H), lambda i: (i, 0))],
        out_shape=[jax.ShapeDtypeStruct((PAR_P, 3 * H), f32),
                   jax.ShapeDtypeStruct((PAR_P, H), f32)],
    )(x_buf, W_iou, b_iou2, W_f, b_f2)

    h3, c3 = pl.pallas_call(
        _cell3_body,
        grid=(P3 // 256,),
        in_specs=[pl.BlockSpec((256, H), lambda i: (PAR_P // 256 + i, 0)),
                  _rep((H, 3 * H)), _rep((1, 3 * H))],
        out_specs=[pl.BlockSpec((256, H), lambda i: (i, 0))] * 2,
        out_shape=[jax.ShapeDtypeStruct((P3, H), f32)] * 2,
    )(x_buf, W_iou, b_iou2)

    def up_level(h_child, c_child, par_g, par_s, n_child, n_chunks, ch,
                 n_seg, sp_rows, xiou_block_off, n_l):
        xpf = _make_gather(n_child, n_chunks, ch, (PAR_P, H))(par_g, xwf_par)
        blk = min(256, n_child)
        fc = pl.pallas_call(
            _f_body,
            grid=(n_child // blk,),
            in_specs=[pl.BlockSpec((blk, H), lambda i: (i, 0))] * 3 + [_rep((H, H))],
            out_specs=pl.BlockSpec((blk, H), lambda i: (i, 0)),
            out_shape=jax.ShapeDtypeStruct((n_child, H), f32),
        )(xpf, h_child, c_child, U_f)
        hs, fcs = _make_scatter(n_child, n_chunks, ch, n_seg, sp_rows)(
            par_s, h_child, fc, zeros_sp)
        h, c = pl.pallas_call(
            _cell_body,
            grid=(n_l // 64,),
            in_specs=[pl.BlockSpec((64, 3 * H), lambda i, o=xiou_block_off: (o + i, 0)),
                      pl.BlockSpec((NC, 64, H), lambda i: (0, i, 0)),
                      pl.BlockSpec((NC, 64, H), lambda i: (0, i, 0)),
                      _rep((H, 3 * H))],
            out_specs=[pl.BlockSpec((64, H), lambda i: (i, 0))] * 2,
            out_shape=[jax.ShapeDtypeStruct((n_l, H), f32)] * 2,
        )(xiou_par, hs, fcs, U_iou)
        return h, c

    h2, c2 = up_level(h3, c3, par2_g, par2_s, P3, 13, 112, 4096, 4224,
                      STARTS[2] // 64, 4096)
    h1, c1 = up_level(h2, c2, 64 + par1, par1, 4096, 1, 128, 512, 640,
                      STARTS[1] // 64, 512)
    h0, _ = up_level(h1, c1, par0, par0, 512, 1, 16, 64, 128,
                     STARTS[0] // 64, 64)
    return h0


# speedup vs baseline: 1.3148x; 1.0267x over previous
import functools

import jax
import jax.numpy as jnp
from jax import lax
from jax.experimental import pallas as pl
from jax.experimental.pallas import tpu as pltpu
from jax.experimental.pallas import tpu_sc as plsc

H = 128
NC, NS = 2, 16
NW = NC * NS

LEVEL_SIZES = [64, 512, 4096, 45328]
STARTS = [0, 64, 576, 4672]

PAR_P = 4864
P3 = 46592
B_G = 53248
XPF_G = 53248

_mesh = plsc.VectorSubcoreMesh(
    core_axis_name="c", subcore_axis_name="s", num_cores=NC, num_subcores=NS)


def _make_gather(n_chunks, ch, table_shape, nbuf=4):
    n_rows = NW * n_chunks * ch
    nbuf = min(nbuf, n_chunks)

    @functools.partial(
        pl.kernel, mesh=_mesh,
        out_type=jax.ShapeDtypeStruct((n_rows, H), jnp.float32),
        scratch_types=[
            pltpu.VMEM((n_chunks, ch), jnp.int32),
            pltpu.VMEM((nbuf, ch, H), jnp.float32),
        ] + [pltpu.SemaphoreType.DMA] * (2 * nbuf),
    )
    def gather_k(idx_hbm, table_hbm, out_hbm, idx_v, bufs, *sems):
        gsem, wsem = sems[:nbuf], sems[nbuf:]
        wid = lax.axis_index("c") * NS + lax.axis_index("s")
        base = wid * (n_chunks * ch)
        pltpu.sync_copy(idx_hbm.at[wid], idx_v)
        gd = [None] * n_chunks
        wd = [None] * n_chunks
        for j in range(nbuf):
            gd[j] = pltpu.async_copy(
                table_hbm.at[idx_v.at[j]], bufs.at[j], gsem[j])
        for j in range(n_chunks):
            b = j % nbuf
            gd[j].wait()
            wd[j] = pltpu.async_copy(
                bufs.at[b], out_hbm.at[pl.ds(base + j * ch, ch)], wsem[b])
            nj = j + nbuf
            if nj < n_chunks:
                wd[j].wait()
                gd[nj] = pltpu.async_copy(
                    table_hbm.at[idx_v.at[nj]], bufs.at[b], gsem[b])
        for j in range(max(0, n_chunks - nbuf), n_chunks):
            wd[j].wait()

    return gather_k


def _make_scatter(n_chunks, ch, n_seg, sp_rows, nbuf=2):
    n_rows = NW * n_chunks * ch
    nbuf = min(nbuf, n_chunks)
    assert sp_rows % (8 * NS) == 0 and n_seg % 8 == 0
    zr = sp_rows // NS
    out_tiles = min(NS, n_seg // 8)
    orows = n_seg // out_tiles

    @functools.partial(
        pl.kernel, mesh=_mesh,
        out_type=(jax.ShapeDtypeStruct((NC, n_seg, H), jnp.float32),
                  jax.ShapeDtypeStruct((NC, n_seg, H), jnp.float32)),
        scratch_types=[
            pltpu.VMEM((n_chunks, ch), jnp.int32),
            pltpu.VMEM((nbuf, ch, H), jnp.float32),
            pltpu.VMEM((nbuf, ch, H), jnp.float32),
            pltpu.VMEM_SHARED((sp_rows, H), jnp.float32),
            pltpu.VMEM_SHARED((sp_rows, H), jnp.float32),
        ] + [pltpu.SemaphoreType.DMA] * (2 * nbuf),
    )
    def scatter_k(idx_hbm, h_hbm, fc_hbm, zeros_hbm, hs_out, fcs_out,
                  idx_v, hbuf, fbuf, hsum_sh, fcsum_sh, *sems):
        hsem, fsem = sems[:nbuf], sems[nbuf:]
        c = lax.axis_index("c")
        s = lax.axis_index("s")
        wid = c * NS + s
        base = wid * (n_chunks * ch)
        pltpu.sync_copy(zeros_hbm.at[pl.ds(s * zr, zr)], hsum_sh.at[pl.ds(s * zr, zr)])
        pltpu.sync_copy(zeros_hbm.at[pl.ds(s * zr, zr)], fcsum_sh.at[pl.ds(s * zr, zr)])
        pltpu.sync_copy(idx_hbm.at[wid], idx_v)
        plsc.subcore_barrier()
        hd = [None] * n_chunks
        fd = [None] * n_chunks
        for j in range(nbuf):
            off = base + j * ch
            hd[j] = pltpu.async_copy(h_hbm.at[pl.ds(off, ch)], hbuf.at[j], hsem[j])
            fd[j] = pltpu.async_copy(fc_hbm.at[pl.ds(off, ch)], fbuf.at[j], fsem[j])
        for j in range(n_chunks):
            b = j % nbuf
            hd[j].wait()
            fd[j].wait()
            pltpu.sync_copy(hbuf.at[b], hsum_sh.at[idx_v.at[j]], add=True)
            pltpu.sync_copy(fbuf.at[b], fcsum_sh.at[idx_v.at[j]], add=True)
            nj = j + nbuf
            if nj < n_chunks:
                off = base + nj * ch
                hd[nj] = pltpu.async_copy(h_hbm.at[pl.ds(off, ch)], hbuf.at[b], hsem[b])
                fd[nj] = pltpu.async_copy(fc_hbm.at[pl.ds(off, ch)], fbuf.at[b], fsem[b])
        plsc.subcore_barrier()

        @pl.when(s < out_tiles)
        def _():
            pltpu.sync_copy(hsum_sh.at[pl.ds(s * orows, orows)],
                            hs_out.at[c, pl.ds(s * orows, orows)])
            pltpu.sync_copy(fcsum_sh.at[pl.ds(s * orows, orows)],
                            fcs_out.at[c, pl.ds(s * orows, orows)])

    return scatter_k


def _xw_body(x_ref, wiou_ref, biou_ref, wf_ref, bf_ref, xiou_ref, xwf_ref):
    x = x_ref[...]
    xiou_ref[...] = jnp.dot(x, wiou_ref[...], preferred_element_type=jnp.float32) + biou_ref[...]
    xwf_ref[...] = jnp.dot(x, wf_ref[...], preferred_element_type=jnp.float32) + bf_ref[...]


def _cell3_body(x_ref, wiou_ref, biou_ref, h_ref, c_ref):
    iou = jnp.dot(x_ref[...], wiou_ref[...], preferred_element_type=jnp.float32) + biou_ref[...]
    i = jax.nn.sigmoid(iou[:, :H])
    o = jax.nn.sigmoid(iou[:, H:2 * H])
    u = jnp.tanh(iou[:, 2 * H:])
    c = i * u
    c_ref[...] = c
    h_ref[...] = o * jnp.tanh(c)


def _f_body(xpf_ref, h_ref, c_ref, uf_ref, fc_ref):
    pre = xpf_ref[...] + jnp.dot(h_ref[...], uf_ref[...], preferred_element_type=jnp.float32)
    fc_ref[...] = jax.nn.sigmoid(pre) * c_ref[...]


def _cell_body(xiou_ref, hs_ref, fcs_ref, uiou_ref, h_ref, c_ref):
    h_sum = hs_ref[0] + hs_ref[1]
    iou = xiou_ref[...] + jnp.dot(h_sum, uiou_ref[...], preferred_element_type=jnp.float32)
    i = jax.nn.sigmoid(iou[:, :H])
    o = jax.nn.sigmoid(iou[:, H:2 * H])
    u = jnp.tanh(iou[:, 2 * H:])
    c = i * u + (fcs_ref[0] + fcs_ref[1])
    c_ref[...] = c
    h_ref[...] = o * jnp.tanh(c)


def _rep(shape):
    return pl.BlockSpec(shape, lambda i: tuple(0 for _ in shape))


def kernel(token_ids, parent_raw, emb, W_iou, U_iou, b_iou, W_f, U_f, b_f):
    f32 = jnp.float32
    token_ids = token_ids.astype(jnp.int32)
    parent_raw = parent_raw.astype(jnp.int32)
    b_iou2 = b_iou.reshape(1, 3 * H)
    b_f2 = b_f.reshape(1, H)

    tid = jnp.concatenate([
        token_ids[:4672], jnp.zeros((PAR_P - 4672,), jnp.int32),
        token_ids[4672:], jnp.zeros((B_G - PAR_P - 45328,), jnp.int32)])
    pad2 = P3 - 45328
    par2 = parent_raw[4672:50000] % 4096
    par2_g = jnp.concatenate([576 + par2, jnp.full((pad2,), 4672, jnp.int32)])
    par2_s = jnp.concatenate([par2, jnp.full((pad2,), 4096, jnp.int32)])
    par1 = parent_raw[576:4672] % 512
    par0 = parent_raw[64:576] % 64
    parg_all = jnp.concatenate([
        par2_g, 64 + par1, par0, jnp.zeros((XPF_G - 51200,), jnp.int32)])
    zeros_sp = jnp.zeros((4224, H), f32)

    x_buf = _make_gather(13, 128, emb.shape)(tid.reshape(NW, 13, 128), emb)

    xiou_par, xwf_par = pl.pallas_call(
        _xw_body,
        grid=(PAR_P // 128,),
        in_specs=[pl.BlockSpec((128, H), lambda i: (i, 0)),
                  _rep((H, 3 * H)), _rep((1, 3 * H)),
                  _rep((H, H)), _rep((1, H))],
        out_specs=[pl.BlockSpec((128, 3 * H), lambda i: (i, 0)),
                   pl.BlockSpec((128, H), lambda i: (i, 0))],
        out_shape=[jax.ShapeDtypeStruct((PAR_P, 3 * H), f32),
                   jax.ShapeDtypeStruct((PAR_P, H), f32)],
    )(x_buf, W_iou, b_iou2, W_f, b_f2)

    xpf_all = _make_gather(13, 128, (PAR_P, H))(
        parg_all.reshape(NW, 13, 128), xwf_par)

    h3, c3 = pl.pallas_call(
        _cell3_body,
        grid=(P3 // 256,),
        in_specs=[pl.BlockSpec((256, H), lambda i: (PAR_P // 256 + i, 0)),
                  _rep((H, 3 * H)), _rep((1, 3 * H))],
        out_specs=[pl.BlockSpec((256, H), lambda i: (i, 0))] * 2,
        out_shape=[jax.ShapeDtypeStruct((P3, H), f32)] * 2,
    )(x_buf, W_iou, b_iou2)

    def up_level(h_child, c_child, par_s, n_child, n_chunks, ch,
                 n_seg, sp_rows, xpf_block_off, xiou_block_off, n_l):
        blk = min(256, n_child)
        fc = pl.pallas_call(
            _f_body,
            grid=(n_child // blk,),
            in_specs=[pl.BlockSpec((blk, H), lambda i, o=xpf_block_off: (o + i, 0))]
                     + [pl.BlockSpec((blk, H), lambda i: (i, 0))] * 2
                     + [_rep((H, H))],
            out_specs=pl.BlockSpec((blk, H), lambda i: (i, 0)),
            out_shape=jax.ShapeDtypeStruct((n_child, H), f32),
        )(xpf_all, h_child, c_child, U_f)
        hs, fcs = _make_scatter(n_chunks, ch, n_seg, sp_rows)(
            par_s.reshape(NW, n_chunks, ch), h_child, fc, zeros_sp)
        h, c = pl.pallas_call(
            _cell_body,
            grid=(n_l // 64,),
            in_specs=[pl.BlockSpec((64, 3 * H), lambda i, o=xiou_block_off: (o + i, 0)),
                      pl.BlockSpec((NC, 64, H), lambda i: (0, i, 0)),
                      pl.BlockSpec((NC, 64, H), lambda i: (0, i, 0)),
                      _rep((H, 3 * H))],
            out_specs=[pl.BlockSpec((64, H), lambda i: (i, 0))] * 2,
            out_shape=[jax.ShapeDtypeStruct((n_l, H), f32)] * 2,
        )(xiou_par, hs, fcs, U_iou)
        return h, c

    h2, c2 = up_level(h3, c3, par2_s, P3, 13, 112, 4096, 4224,
                      0, STARTS[2] // 64, 4096)
    h1, c1 = up_level(h2, c2, par1, 4096, 1, 128, 512, 640,
                      P3 // 256, STARTS[1] // 64, 512)
    h0, _ = up_level(h1, c1, par0, 512, 1, 16, 64, 128,
                     (P3 + 4096) // 256, STARTS[0] // 64, 64)
    return h0


# speedup vs baseline: 1.8488x; 1.4062x over previous
import functools

import jax
import jax.numpy as jnp
from jax import lax
from jax.experimental import pallas as pl
from jax.experimental.pallas import tpu as pltpu
from jax.experimental.pallas import tpu_sc as plsc

H = 128
NC, NS = 2, 16
NW = NC * NS

PAR_P = 5120
L2_OFF, L1_OFF, L0_OFF = 0, 4096, 4608
P3 = 46592
B_G = 53248
XPF_G = 51200

_mesh = plsc.VectorSubcoreMesh(
    core_axis_name="c", subcore_axis_name="s", num_cores=NC, num_subcores=NS)


def _make_gather(n_chunks, ch, dtype, nbuf=4):
    n_rows = NW * n_chunks * ch
    nbuf = min(nbuf, n_chunks)

    @functools.partial(
        pl.kernel, mesh=_mesh,
        out_type=jax.ShapeDtypeStruct((n_rows, H), dtype),
        scratch_types=[
            pltpu.VMEM((n_chunks, ch), jnp.int32),
            pltpu.VMEM((nbuf, ch, H), dtype),
        ] + [pltpu.SemaphoreType.DMA] * (2 * nbuf),
    )
    def gather_k(idx_hbm, table_hbm, out_hbm, idx_v, bufs, *sems):
        gsem, wsem = sems[:nbuf], sems[nbuf:]
        wid = lax.axis_index("c") * NS + lax.axis_index("s")
        base = wid * (n_chunks * ch)
        pltpu.sync_copy(idx_hbm.at[wid], idx_v)
        gd = [None] * n_chunks
        wd = [None] * n_chunks
        for j in range(nbuf):
            gd[j] = pltpu.async_copy(
                table_hbm.at[idx_v.at[j]], bufs.at[j], gsem[j])
        for j in range(n_chunks):
            b = j % nbuf
            gd[j].wait()
            wd[j] = pltpu.async_copy(
                bufs.at[b], out_hbm.at[pl.ds(base + j * ch, ch)], wsem[b])
            nj = j + nbuf
            if nj < n_chunks:
                wd[j].wait()
                gd[nj] = pltpu.async_copy(
                    table_hbm.at[idx_v.at[nj]], bufs.at[b], gsem[b])
        for j in range(max(0, n_chunks - nbuf), n_chunks):
            wd[j].wait()

    return gather_k


def _make_scatter(n_chunks, ch, n_seg, sp_rows, nbuf=2):
    nbuf = min(nbuf, n_chunks)
    assert sp_rows % (8 * NS) == 0 and n_seg % 8 == 0
    zr = sp_rows // NS
    out_tiles = min(NS, n_seg // 8)
    orows = n_seg // out_tiles

    @functools.partial(
        pl.kernel, mesh=_mesh,
        out_type=(jax.ShapeDtypeStruct((NC, n_seg, H), jnp.float32),
                  jax.ShapeDtypeStruct((NC, n_seg, H), jnp.float32)),
        scratch_types=[
            pltpu.VMEM((n_chunks, ch), jnp.int32),
            pltpu.VMEM((nbuf, ch, H), jnp.float32),
            pltpu.VMEM((nbuf, ch, H), jnp.float32),
            pltpu.VMEM_SHARED((sp_rows, H), jnp.float32),
            pltpu.VMEM_SHARED((sp_rows, H), jnp.float32),
        ] + [pltpu.SemaphoreType.DMA] * (2 * nbuf),
    )
    def scatter_k(idx_hbm, h_hbm, fc_hbm, zeros_hbm, hs_out, fcs_out,
                  idx_v, hbuf, fbuf, hsum_sh, fcsum_sh, *sems):
        hsem, fsem = sems[:nbuf], sems[nbuf:]
        c = lax.axis_index("c")
        s = lax.axis_index("s")
        wid = c * NS + s
        base = wid * (n_chunks * ch)
        pltpu.sync_copy(zeros_hbm.at[pl.ds(s * zr, zr)], hsum_sh.at[pl.ds(s * zr, zr)])
        pltpu.sync_copy(zeros_hbm.at[pl.ds(s * zr, zr)], fcsum_sh.at[pl.ds(s * zr, zr)])
        pltpu.sync_copy(idx_hbm.at[wid], idx_v)
        plsc.subcore_barrier()
        hd = [None] * n_chunks
        fd = [None] * n_chunks
        for j in range(nbuf):
            off = base + j * ch
            hd[j] = pltpu.async_copy(h_hbm.at[pl.ds(off, ch)], hbuf.at[j], hsem[j])
            fd[j] = pltpu.async_copy(fc_hbm.at[pl.ds(off, ch)], fbuf.at[j], fsem[j])
        for j in range(n_chunks):
            b = j % nbuf
            hd[j].wait()
            fd[j].wait()
            pltpu.sync_copy(hbuf.at[b], hsum_sh.at[idx_v.at[j]], add=True)
            pltpu.sync_copy(fbuf.at[b], fcsum_sh.at[idx_v.at[j]], add=True)
            nj = j + nbuf
            if nj < n_chunks:
                off = base + nj * ch
                hd[nj] = pltpu.async_copy(h_hbm.at[pl.ds(off, ch)], hbuf.at[b], hsem[b])
                fd[nj] = pltpu.async_copy(fc_hbm.at[pl.ds(off, ch)], fbuf.at[b], fsem[b])
        plsc.subcore_barrier()

        @pl.when(s < out_tiles)
        def _():
            pltpu.sync_copy(hsum_sh.at[pl.ds(s * orows, orows)],
                            hs_out.at[c, pl.ds(s * orows, orows)])
            pltpu.sync_copy(fcsum_sh.at[pl.ds(s * orows, orows)],
                            fcs_out.at[c, pl.ds(s * orows, orows)])

    return scatter_k


def _xw_body(x_ref, wiou_ref, biou_ref, wf_ref, bf_ref, xiou_ref, xwf_ref):
    x = x_ref[...]
    xiou_ref[...] = jnp.dot(x, wiou_ref[...], preferred_element_type=jnp.float32) + biou_ref[...]
    xwf_ref[...] = jnp.dot(x, wf_ref[...], preferred_element_type=jnp.float32) + bf_ref[...]


def _cellf3_body(x_ref, wiou_ref, biou_ref, xpf_ref, uf_ref, h_ref, fc_ref):
    iou = jnp.dot(x_ref[...], wiou_ref[...], preferred_element_type=jnp.float32) + biou_ref[...]
    i = jax.nn.sigmoid(iou[:, :H])
    o = jax.nn.sigmoid(iou[:, H:2 * H])
    u = jnp.tanh(iou[:, 2 * H:])
    c = i * u
    h = o * jnp.tanh(c)
    h_ref[...] = h
    pre = xpf_ref[...].astype(jnp.float32) + jnp.dot(
        h, uf_ref[...], preferred_element_type=jnp.float32)
    fc_ref[...] = jax.nn.sigmoid(pre) * c


def _cellf_body(xiou_ref, hs_ref, fcs_ref, uiou_ref, xpf_ref, uf_ref,
                h_ref, fc_ref):
    h_sum = hs_ref[0] + hs_ref[1]
    iou = xiou_ref[...] + jnp.dot(h_sum, uiou_ref[...], preferred_element_type=jnp.float32)
    i = jax.nn.sigmoid(iou[:, :H])
    o = jax.nn.sigmoid(iou[:, H:2 * H])
    u = jnp.tanh(iou[:, 2 * H:])
    c = i * u + (fcs_ref[0] + fcs_ref[1])
    h = o * jnp.tanh(c)
    h_ref[...] = h
    pre = xpf_ref[...].astype(jnp.float32) + jnp.dot(
        h, uf_ref[...], preferred_element_type=jnp.float32)
    fc_ref[...] = jax.nn.sigmoid(pre) * c


def _cell0_body(xiou_ref, hs_ref, fcs_ref, uiou_ref, h_ref):
    h_sum = hs_ref[0] + hs_ref[1]
    iou = xiou_ref[...] + jnp.dot(h_sum, uiou_ref[...], preferred_element_type=jnp.float32)
    i = jax.nn.sigmoid(iou[:, :H])
    o = jax.nn.sigmoid(iou[:, H:2 * H])
    u = jnp.tanh(iou[:, 2 * H:])
    c = i * u + (fcs_ref[0] + fcs_ref[1])
    h_ref[...] = o * jnp.tanh(c)


def _rep(shape):
    return pl.BlockSpec(shape, lambda i: tuple(0 for _ in shape))


def kernel(token_ids, parent_raw, emb, W_iou, U_iou, b_iou, W_f, U_f, b_f):
    f32 = jnp.float32
    token_ids = token_ids.astype(jnp.int32)
    parent_raw = parent_raw.astype(jnp.int32)
    b_iou2 = b_iou.reshape(1, 3 * H)
    b_f2 = b_f.reshape(1, H)

    tid = jnp.concatenate([
        token_ids[576:4672], token_ids[64:576], token_ids[:64],
        jnp.zeros((PAR_P - 4672,), jnp.int32),
        token_ids[4672:], jnp.zeros((B_G - PAR_P - 45328,), jnp.int32)])
    pad2 = P3 - 45328
    par2 = parent_raw[4672:50000] % 4096
    par2_g = jnp.concatenate([L2_OFF + par2, jnp.full((pad2,), 4672, jnp.int32)])
    par2_s = jnp.concatenate([par2, jnp.full((pad2,), 4096, jnp.int32)])
    par1 = parent_raw[576:4672] % 512
    par0 = parent_raw[64:576] % 64
    parg_all = jnp.concatenate([par2_g, L1_OFF + par1, L0_OFF + par0])
    zeros_sp = jnp.zeros((4224, H), f32)

    x_buf = _make_gather(13, 128, f32)(tid.reshape(NW, 13, 128), emb)

    xiou_par, xwf_par = pl.pallas_call(
        _xw_body,
        grid=(PAR_P // 256,),
        in_specs=[pl.BlockSpec((256, H), lambda i: (i, 0)),
                  _rep((H, 3 * H)), _rep((1, 3 * H)),
                  _rep((H, H)), _rep((1, H))],
        out_specs=[pl.BlockSpec((256, 3 * H), lambda i: (i, 0)),
                   pl.BlockSpec((256, H), lambda i: (i, 0))],
        out_shape=[jax.ShapeDtypeStruct((PAR_P, 3 * H), f32),
                   jax.ShapeDtypeStruct((PAR_P, H), f32)],
    )(x_buf, W_iou, b_iou2, W_f, b_f2)

    xpf_all = _make_gather(20, 80, f32)(
        parg_all.reshape(NW, 20, 80), xwf_par)

    h3, fc2 = pl.pallas_call(
        _cellf3_body,
        grid=(P3 // 512,),
        in_specs=[pl.BlockSpec((512, H), lambda i: (PAR_P // 512 + i, 0)),
                  _rep((H, 3 * H)), _rep((1, 3 * H)),
                  pl.BlockSpec((512, H), lambda i: (i, 0)),
                  _rep((H, H))],
        out_specs=[pl.BlockSpec((512, H), lambda i: (i, 0))] * 2,
        out_shape=[jax.ShapeDtypeStruct((P3, H), f32)] * 2,
    )(x_buf, W_iou, b_iou2, xpf_all, U_f)

    def cellf(hs, fcs, n_l, xiou_off, xpf_off):
        blk = min(512, n_l)
        return pl.pallas_call(
            _cellf_body,
            grid=(n_l // blk,),
            in_specs=[pl.BlockSpec((blk, 3 * H), lambda i, o=xiou_off // blk: (o + i, 0)),
                      pl.BlockSpec((NC, blk, H), lambda i: (0, i, 0)),
                      pl.BlockSpec((NC, blk, H), lambda i: (0, i, 0)),
                      _rep((H, 3 * H)),
                      pl.BlockSpec((blk, H), lambda i, o=xpf_off // blk: (o + i, 0)),
                      _rep((H, H))],
            out_specs=[pl.BlockSpec((blk, H), lambda i: (i, 0))] * 2,
            out_shape=[jax.ShapeDtypeStruct((n_l, H), f32)] * 2,
        )(xiou_par, hs, fcs, U_iou, xpf_all, U_f)

    hs2, fcs2 = _make_scatter(13, 112, 4096, 4224)(
        par2_s.reshape(NW, 13, 112), h3, fc2, zeros_sp)
    h2, fc1 = cellf(hs2, fcs2, 4096, L2_OFF, P3)
    hs1, fcs1 = _make_scatter(1, 128, 512, 640)(
        par1.reshape(NW, 1, 128), h2, fc1, zeros_sp)
    h1, fc0 = cellf(hs1, fcs1, 512, L1_OFF, P3 + 4096)
    hs0, fcs0 = _make_scatter(1, 16, 64, 128)(
        par0.reshape(NW, 1, 16), h1, fc0, zeros_sp)
    h0 = pl.pallas_call(
        _cell0_body,
        grid=(1,),
        in_specs=[pl.BlockSpec((64, 3 * H), lambda i: (L0_OFF // 64 + i, 0)),
                  pl.BlockSpec((NC, 64, H), lambda i: (0, i, 0)),
                  pl.BlockSpec((NC, 64, H), lambda i: (0, i, 0)),
                  _rep((H, 3 * H))],
        out_specs=pl.BlockSpec((64, H), lambda i: (i, 0)),
        out_shape=jax.ShapeDtypeStruct((64, H), f32),
    )(xiou_par, hs0, fcs0, U_iou)
    return h0
